# Initial kernel scaffold; baseline (speedup 1.0000x reference)
import jax
import jax.numpy as jnp
from jax.experimental import pallas as pl


def kernel(x_position, edge_index, edge_attr, fc0_w, fc0_b, Wih, Whh, bih, bhh, kw1, kb1, kw2, kb2, kw3, kb3, root1, bias1, root2, bias2, fc3_w, fc3_b):
    raise NotImplementedError("write your pallas kernel here")


# speedup vs baseline: 2.4240x; 2.4240x over previous
import functools

import jax
import jax.numpy as jnp
from jax import lax
from jax.experimental import pallas as pl
from jax.experimental.pallas import tpu as pltpu
from jax.experimental.pallas import tpu_sc as plsc

B = 1250
T = 8
NN = 8
D = 3
ND = 24
E = 160000
EP = 163840
N = 10000
NP = 10016
KER_IN = 16
KW = 32
BPAD = 1280

NWORK = 32
EDGES_PER_W = EP // NWORK
CH = 1024
NCH = EDGES_PER_W // CH
RPC = CH // 128
IDXROWS = EP // 128
ACC_STRIPE = NP // 16

_f32 = jnp.float32


def _lstm_body(x_ref, fc0w_ref, fc0b_ref, wg_ref, ug_ref, bg_ref, out_ref):
    fc0w = fc0w_ref[...]
    fc0b = fc0b_ref[...]
    wg = wg_ref[...]
    ug = ug_ref[...]
    bg = bg_ref[...]

    def step(t, carry):
        h, c = carry
        xt = x_ref[t]
        xf = jnp.dot(xt, fc0w, preferred_element_type=_f32) + fc0b
        g = (jnp.dot(xf, wg, preferred_element_type=_f32)
             + jnp.dot(h, ug, preferred_element_type=_f32) + bg)
        i = jax.nn.sigmoid(g[:, 0:ND])
        f = jax.nn.sigmoid(g[:, 128:128 + ND])
        gg = jnp.tanh(g[:, 256:256 + ND])
        o = jax.nn.sigmoid(g[:, 384:384 + ND])
        c2 = f * c + i * gg
        h2 = o * jnp.tanh(c2)
        out_ref[t] = h2
        return (h2, c2)

    z = jnp.zeros((BPAD, ND), _f32)
    lax.fori_loop(0, T, step, (z, z))


_lstm_call = pl.pallas_call(
    _lstm_body,
    out_shape=jax.ShapeDtypeStruct((T, BPAD, ND), _f32),
)


EB = 2048


def _msg_body(ea_ref, xj_ref, kw1_ref, kb1_ref, kw2_ref, kb2_ref,
              k3_ref, qh_ref, rp_ref, kbp_ref, be_ref, out_ref):
    ea = ea_ref[...]
    xj = xj_ref[...]
    h1 = jnp.maximum(jnp.dot(ea, kw1_ref[...], preferred_element_type=_f32)
                     + kb1_ref[...], 0.0)
    h2 = jnp.maximum(jnp.dot(h1, kw2_ref[...], preferred_element_type=_f32)
                     + kb2_ref[...], 0.0)
    v = jnp.dot(xj, k3_ref[...], preferred_element_type=_f32)
    hh = jnp.dot(h2, qh_ref[...], preferred_element_type=_f32)
    m = jnp.dot(v * hh, rp_ref[...], preferred_element_type=_f32)
    m = m + jnp.dot(xj, kbp_ref[...], preferred_element_type=_f32) + be_ref[...]
    out_ref[...] = m


_msg_call = pl.pallas_call(
    _msg_body,
    grid=(EP // EB,),
    in_specs=[
        pl.BlockSpec((EB, KER_IN), lambda i: (i, 0)),
        pl.BlockSpec((EB, 32), lambda i: (i, 0)),
        pl.BlockSpec((KER_IN, KW), lambda i: (0, 0)),
        pl.BlockSpec((1, KW), lambda i: (0, 0)),
        pl.BlockSpec((KW, KW), lambda i: (0, 0)),
        pl.BlockSpec((1, KW), lambda i: (0, 0)),
        pl.BlockSpec((32, KW * ND), lambda i: (0, 0)),
        pl.BlockSpec((KW, KW * ND), lambda i: (0, 0)),
        pl.BlockSpec((KW * ND, 32), lambda i: (0, 0)),
        pl.BlockSpec((32, 32), lambda i: (0, 0)),
        pl.BlockSpec((1, 32), lambda i: (0, 0)),
    ],
    out_specs=pl.BlockSpec((EB, 32), lambda i: (i, 0)),
    out_shape=jax.ShapeDtypeStruct((EP, 32), _f32),
)


def _node_core(acc_ref, x_ref, root_ref, b_ref, mask_ref):
    s = acc_ref[0] + acc_ref[1]
    inv = 1.0 / jnp.maximum(s[:, ND:ND + 1], 1.0)
    aggr = s * inv
    y = aggr + jnp.dot(x_ref[...], root_ref[...], preferred_element_type=_f32)
    return jnp.maximum(y + b_ref[...], 0.0) * mask_ref[...]


def _node_body(acc_ref, x_ref, root_ref, b_ref, mask_ref, out_ref):
    out_ref[...] = _node_core(acc_ref, x_ref, root_ref, b_ref, mask_ref)


def _node_final_body(acc_ref, x_ref, root_ref, b_ref, mask_ref,
                     fc3w_ref, fc3b_ref, out_ref):
    x3 = _node_core(acc_ref, x_ref, root_ref, b_ref, mask_ref)
    out_ref[...] = (jnp.dot(x3, fc3w_ref[...], preferred_element_type=_f32)
                    + fc3b_ref[...])


_node_call = pl.pallas_call(
    _node_body,
    out_shape=jax.ShapeDtypeStruct((NP, 32), _f32),
)

_node_final_call = pl.pallas_call(
    _node_final_body,
    out_shape=jax.ShapeDtypeStruct((NP, 32), _f32),
)


def _gather_body(x_hbm, src_hbm, out_hbm, idx_v, rows_v, sem):
    c = lax.axis_index("c")
    s = lax.axis_index("s")
    wid = s * 2 + c

    def chunk(j, carry):
        ebase = wid * EDGES_PER_W + j * CH
        rbase = wid * (EDGES_PER_W // 128) + j * RPC
        pltpu.sync_copy(src_hbm.at[pl.ds(rbase, RPC)], idx_v)
        for k in range(RPC):
            pltpu.async_copy(x_hbm.at[idx_v.at[k]],
                             rows_v.at[pl.ds(k * 128, 128)], sem).wait()
        pltpu.sync_copy(rows_v, out_hbm.at[pl.ds(ebase, CH)])
        return carry

    lax.fori_loop(0, NCH, chunk, 0)


def _scatter_body(msg_hbm, dst_hbm, zeros_hbm, out_hbm,
                  idx_v, rows_v, stripe_v, acc_sh, sem):
    c = lax.axis_index("c")
    s = lax.axis_index("s")
    wid = s * 2 + c

    pltpu.sync_copy(zeros_hbm, stripe_v)
    pltpu.sync_copy(stripe_v, acc_sh.at[pl.ds(s * ACC_STRIPE, ACC_STRIPE)])
    plsc.subcore_barrier()

    def chunk(j, carry):
        ebase = wid * EDGES_PER_W + j * CH
        rbase = wid * (EDGES_PER_W // 128) + j * RPC
        pltpu.sync_copy(dst_hbm.at[pl.ds(rbase, RPC)], idx_v)
        pltpu.sync_copy(msg_hbm.at[pl.ds(ebase, CH)], rows_v)
        for k in range(RPC):
            pltpu.sync_copy(rows_v.at[pl.ds(k * 128, 128)],
                            acc_sh.at[idx_v.at[k]], add=True)
        return carry

    lax.fori_loop(0, NCH, chunk, 0)
    plsc.subcore_barrier()

    pltpu.sync_copy(acc_sh.at[pl.ds(s * ACC_STRIPE, ACC_STRIPE)], stripe_v)
    pltpu.sync_copy(stripe_v, out_hbm.at[c, pl.ds(s * ACC_STRIPE, ACC_STRIPE)])


@functools.lru_cache(maxsize=None)
def _sc_calls():
    mesh = plsc.VectorSubcoreMesh(core_axis_name="c", subcore_axis_name="s")
    params = pltpu.CompilerParams(use_tc_tiling_on_sc=False)
    gather = pl.kernel(
        _gather_body,
        out_type=jax.ShapeDtypeStruct((EP, 32), _f32),
        mesh=mesh,
        compiler_params=params,
        scratch_types=[
            pltpu.VMEM((RPC, 128), jnp.int32),
            pltpu.VMEM((CH, 32), _f32),
            pltpu.SemaphoreType.DMA,
        ],
    )
    scatter = pl.kernel(
        _scatter_body,
        out_type=jax.ShapeDtypeStruct((2, NP, 32), _f32),
        mesh=mesh,
        compiler_params=params,
        scratch_types=[
            pltpu.VMEM((RPC, 128), jnp.int32),
            pltpu.VMEM((CH, 32), _f32),
            pltpu.VMEM((ACC_STRIPE, 32), _f32),
            pltpu.VMEM_SHARED((NP, 32), _f32),
            pltpu.SemaphoreType.DMA,
        ],
    )
    return gather, scatter


def _gate_pad(wt):
    w = jnp.zeros((ND, 512), _f32)
    for gi in range(4):
        w = w.at[:, gi * 128:gi * 128 + ND].set(wt[:, gi * ND:(gi + 1) * ND])
    return w


def kernel(x_position, edge_index, edge_attr, fc0_w, fc0_b, Wih, Whh, bih, bhh,
           kw1, kb1, kw2, kb2, kw3, kb3, root1, bias1, root2, bias2,
           fc3_w, fc3_b):
    xseq = x_position.reshape(B, T, ND).swapaxes(0, 1)
    xseq = jnp.pad(xseq, ((0, 0), (0, BPAD - B), (0, 0)))
    wg = _gate_pad(Wih.T)
    ug = _gate_pad(Whh.T)
    bsum = bih + bhh
    bg = jnp.zeros((1, 512), _f32)
    for gi in range(4):
        bg = bg.at[0, gi * 128:gi * 128 + ND].set(bsum[gi * ND:(gi + 1) * ND])

    ys = _lstm_call(xseq, fc0_w, fc0_b.reshape(1, ND), wg, ug, bg)
    x1 = (ys[:, :B, :].reshape(T, B, NN, D)
          .transpose(1, 2, 0, 3).reshape(N, ND))
    x1p = jnp.pad(x1, ((0, NP - N), (0, 32 - ND)))

    src = jnp.concatenate(
        [edge_index[0], jnp.zeros((EP - E,), jnp.int32)]).reshape(IDXROWS, 128)
    dst = jnp.concatenate(
        [edge_index[1], jnp.full((EP - E,), N, jnp.int32)]).reshape(IDXROWS, 128)
    eap = jnp.pad(edge_attr, ((0, EP - E), (0, 0)))
    zrows = jnp.zeros((ACC_STRIPE, 32), _f32)

    k3 = kw3.reshape(KW, ND, ND)
    k3all = jnp.pad(k3.transpose(1, 0, 2).reshape(ND, KW * ND), ((0, 8), (0, 0)))
    qh = jnp.kron(jnp.eye(KW, dtype=_f32), jnp.ones((1, ND), _f32))
    rp = jnp.pad(jnp.tile(jnp.eye(ND, dtype=_f32), (KW, 1)), ((0, 0), (0, 8)))
    kbp = jnp.pad(kb3.reshape(ND, ND), ((0, 8), (0, 8)))
    be = jnp.zeros((1, 32), _f32).at[0, ND].set(1.0)
    mask = jnp.zeros((1, 32), _f32).at[0, :ND].set(1.0)
    root1p = jnp.pad(root1, ((0, 8), (0, 8)))
    b1p = jnp.pad(bias1.reshape(1, ND), ((0, 0), (0, 8)))
    root2p = jnp.pad(root2, ((0, 8), (0, 8)))
    b2p = jnp.pad(bias2.reshape(1, ND), ((0, 0), (0, 8)))
    fc3p = jnp.pad(fc3_w, ((0, 8), (0, 8)))
    fc3bp = jnp.pad(fc3_b.reshape(1, ND), ((0, 0), (0, 8)))
    kb1r = kb1.reshape(1, KW)
    kb2r = kb2.reshape(1, KW)

    _gather_call, _scatter_call = _sc_calls()

    xj1 = _gather_call(x1p, src)
    msg1 = _msg_call(eap, xj1, kw1, kb1r, kw2, kb2r, k3all, qh, rp, kbp, be)
    acc1 = _scatter_call(msg1, dst, zrows)
    x2p = _node_call(acc1, x1p, root1p, b1p, mask)

    xj2 = _gather_call(x2p, src)
    msg2 = _msg_call(eap, xj2, kw1, kb1r, kw2, kb2r, k3all, qh, rp, kbp, be)
    acc2 = _scatter_call(msg2, dst, zrows)
    y = _node_final_call(acc2, x2p, root2p, b2p, mask, fc3p, fc3bp)

    return y[:N, :ND].reshape(-1, D)
